# baseline (speedup 1.0000x reference)
import functools

import jax
import jax.numpy as jnp
from jax import lax
from jax.experimental import pallas as pl
from jax.experimental.pallas import tpu as pltpu
from jax.experimental.pallas import tpu_sc as plsc

N = 10000
NPAD = 10240
D = 128
DOUT = 64
E = 320000
CHUNK = 128
CPT = 80
EPAD = 32 * CPT * CHUNK
NCHUNKS = EPAD // CHUNK
IB = 16
NC, NS = 2, 16
RPT = NPAD // NS
CPT_ALL = 160
PAIRS = CPT_ALL // 2

_mesh1 = plsc.VectorSubcoreMesh(core_axis_name="c", subcore_axis_name="s",
                                num_cores=1)

_mesh = plsc.VectorSubcoreMesh(core_axis_name="c", subcore_axis_name="s")


@functools.partial(
    pl.kernel,
    out_type=jax.ShapeDtypeStruct((NC * NPAD,), jnp.float32),
    mesh=_mesh,
    scratch_types=[
        pltpu.VMEM((CPT, CHUNK), jnp.int32),
        pltpu.VMEM((RPT,), jnp.float32),
        pltpu.VMEM((CHUNK,), jnp.float32),
        pltpu.VMEM_SHARED((NPAD,), jnp.float32),
    ],
)
def _sc_degree(dst_hbm, out_hbm, idx_v, zeros_v, ones_v, acc):
    c = lax.axis_index("c")
    s = lax.axis_index("s")
    tile = c * NS + s
    pltpu.sync_copy(dst_hbm.at[pl.ds(pl.multiple_of(tile * CPT, 8), CPT)],
                    idx_v)

    def _fill(i, _):
        zeros_v[pl.ds(i * 16, 16)] = jnp.zeros((16,), jnp.float32)
        return 0

    lax.fori_loop(0, RPT // 16, _fill, 0)
    for j in range(CHUNK // 16):
        ones_v[pl.ds(j * 16, 16)] = jnp.ones((16,), jnp.float32)
    pltpu.sync_copy(zeros_v, acc.at[pl.ds(s * RPT, RPT)])
    plsc.subcore_barrier()

    def _body(j, _):
        pltpu.sync_copy(ones_v, acc.at[idx_v.at[j]], add=True)
        return 0

    lax.fori_loop(0, CPT, _body, 0)
    plsc.subcore_barrier()
    pltpu.sync_copy(acc.at[pl.ds(s * RPT, RPT)],
                    out_hbm.at[pl.ds(c * NPAD + s * RPT, RPT)])


@functools.partial(
    pl.kernel,
    out_type=jax.ShapeDtypeStruct((NPAD, D), jnp.float32),
    mesh=_mesh1,
    scratch_types=[
        pltpu.VMEM((2, IB, CHUNK), jnp.int32),
        pltpu.VMEM((2, IB, CHUNK), jnp.int32),
        pltpu.VMEM((2, CHUNK, D), jnp.float32),
        pltpu.VMEM_SHARED((NPAD, D), jnp.float32),
        pltpu.SemaphoreType.DMA,
        pltpu.SemaphoreType.DMA,
        pltpu.SemaphoreType.DMA,
        pltpu.SemaphoreType.DMA,
    ],
)
def _sc_adj_apply(y_hbm, src_hbm, dst_hbm, out_hbm,
                  srcb, dstb, rows_v, acc, g0, g1, ssem, isem):
    s = lax.axis_index("s")
    base = s * CPT_ALL

    def _iblk(nb):
        return pl.ds(pl.multiple_of(base + nb * IB, 8), IB)

    pltpu.sync_copy(src_hbm.at[_iblk(0)], srcb.at[0])
    pltpu.sync_copy(dst_hbm.at[_iblk(0)], dstb.at[0])

    def _fill(i, _):
        for j in range(D // 16):
            rows_v[0, i, pl.ds(j * 16, 16)] = jnp.zeros((16,), jnp.float32)
        return 0

    lax.fori_loop(0, CHUNK, _fill, 0)
    for k in range(RPT // CHUNK):
        pltpu.sync_copy(rows_v.at[0],
                        acc.at[pl.ds(s * RPT + k * CHUNK, CHUNK)])

    def _gather_start(blk, row, buf, sem):
        pltpu.async_copy(y_hbm.at[srcb.at[blk, row]], rows_v.at[buf], sem)

    def _gather_wait(blk, row, buf, sem):
        pltpu.make_async_copy(y_hbm.at[srcb.at[blk, row]], rows_v.at[buf],
                              sem).wait()

    _gather_start(0, 0, 0, g0)
    plsc.subcore_barrier()

    def _body(t, _):
        a = 2 * t
        blk = (a // IB) % 2
        ra = a % IB

        @pl.when(jnp.logical_and(t % (IB // 2) == 0, t < PAIRS - IB // 2))
        def _():
            nxt = t // (IB // 2) + 1
            pltpu.async_copy(src_hbm.at[_iblk(nxt)], srcb.at[nxt % 2], isem)
            pltpu.async_copy(dst_hbm.at[_iblk(nxt)], dstb.at[nxt % 2], isem)

        _gather_wait(blk, ra, 0, g0)
        _gather_start(blk, ra + 1, 1, g1)
        sd_a = pltpu.async_copy(rows_v.at[0], acc.at[dstb.at[blk, ra]],
                                ssem, add=True)
        _gather_wait(blk, ra + 1, 1, g1)
        sd_a.wait()

        @pl.when(t < PAIRS - 1)
        def _():
            n = a + 2
            nblk = (n // IB) % 2
            nr = n % IB

            @pl.when(nr == 0)
            def _():
                pltpu.make_async_copy(src_hbm.at[_iblk(0)],
                                      srcb.at[nblk], isem).wait()
                pltpu.make_async_copy(dst_hbm.at[_iblk(0)],
                                      dstb.at[nblk], isem).wait()

            _gather_start(nblk, nr, 0, g0)

        pltpu.sync_copy(rows_v.at[1], acc.at[dstb.at[blk, ra + 1]],
                        add=True)
        return 0

    lax.fori_loop(0, PAIRS, _body, 0)
    plsc.subcore_barrier()
    pltpu.sync_copy(acc.at[pl.ds(s * RPT, RPT)],
                    out_hbm.at[pl.ds(s * RPT, RPT)])


BM = 1024
GRID = NPAD // BM


def _dinv(degp_ref):
    return lax.rsqrt(jnp.maximum(degp_ref[0] + degp_ref[1], 1.0))


def _prep_body(x_ref, w_ref, degp_ref, o_ref):
    o_ref[...] = jnp.dot(x_ref[...], w_ref[...],
                         preferred_element_type=jnp.float32) * _dinv(degp_ref)


def _mid_body(p_ref, degp_ref, b_ref, w_ref, o_ref):
    dinv = _dinv(degp_ref)
    h = jnp.maximum(p_ref[...] * dinv + b_ref[...], 0.0)
    o_ref[...] = jnp.dot(h, w_ref[...],
                         preferred_element_type=jnp.float32) * dinv


def _head_body(p_ref, degp_ref, b2_ref, wf1_ref, bf1_ref, wf2_ref, bf2_ref,
               o_ref):
    dinv = _dinv(degp_ref)
    h = jnp.maximum(p_ref[...] * dinv + b2_ref[...], 0.0)
    f = jnp.maximum(jnp.dot(h, wf1_ref[...],
                            preferred_element_type=jnp.float32) + bf1_ref[...],
                    0.0)
    o_ref[...] = jnp.dot(f, wf2_ref[...],
                         preferred_element_type=jnp.float32) + bf2_ref[...]


_row_spec = pl.BlockSpec((BM, D), lambda i: (i, 0))
_degp_spec = pl.BlockSpec((NC, BM, 1), lambda i: (0, i, 0))
_p_spec = pl.BlockSpec((BM, D), lambda i: (i, 0))
_w_spec = pl.BlockSpec((D, D), lambda i: (0, 0))
_b_spec = pl.BlockSpec((1, D), lambda i: (0, 0))

_tc_prep = pl.pallas_call(
    _prep_body,
    grid=(GRID,),
    in_specs=[_row_spec, _w_spec, _degp_spec],
    out_specs=_row_spec,
    out_shape=jax.ShapeDtypeStruct((NPAD, D), jnp.float32),
)

_tc_mid = pl.pallas_call(
    _mid_body,
    grid=(GRID,),
    in_specs=[_p_spec, _degp_spec, _b_spec, _w_spec],
    out_specs=_row_spec,
    out_shape=jax.ShapeDtypeStruct((NPAD, D), jnp.float32),
)

_tc_head = pl.pallas_call(
    _head_body,
    grid=(GRID,),
    in_specs=[_p_spec, _degp_spec, _b_spec, _w_spec, _b_spec,
              pl.BlockSpec((D, DOUT), lambda i: (0, 0)),
              pl.BlockSpec((1, DOUT), lambda i: (0, 0))],
    out_specs=pl.BlockSpec((BM, DOUT), lambda i: (i, 0)),
    out_shape=jax.ShapeDtypeStruct((NPAD, DOUT), jnp.float32),
)


def kernel(x, edge_index, W1, b1, W2, b2, Wf1, bf1, Wf2, bf2):
    src = edge_index[0].astype(jnp.int32)
    dst = edge_index[1].astype(jnp.int32)
    pad = jnp.full((EPAD - E,), N, jnp.int32)
    src2d = jnp.concatenate([src, pad]).reshape(NCHUNKS, CHUNK)
    dst2d = jnp.concatenate([dst, pad]).reshape(NCHUNKS, CHUNK)
    x_pad = jnp.zeros((NPAD, D), jnp.float32).at[:N].set(x)

    degp = _sc_degree(dst2d).reshape(NC, NPAD)[:, :, None]
    y1 = _tc_prep(x_pad, W1, degp)
    p1 = _sc_adj_apply(y1, src2d, dst2d)
    y2 = _tc_mid(p1, degp, b1.reshape(1, D), W2)
    p2 = _sc_adj_apply(y2, src2d, dst2d)
    out = _tc_head(p2, degp, b2.reshape(1, D), Wf1, bf1.reshape(1, D),
                   Wf2, bf2.reshape(1, DOUT))
    return out[:N]


# speedup vs baseline: 1.2838x; 1.2838x over previous
import functools

import jax
import jax.numpy as jnp
from jax import lax
from jax.experimental import pallas as pl
from jax.experimental.pallas import tpu as pltpu
from jax.experimental.pallas import tpu_sc as plsc

N = 10000
NPAD = 10240
D = 128
DOUT = 64
E = 320000
CHUNK = 128
CPT = 80
EPAD = 32 * CPT * CHUNK
NCHUNKS = EPAD // CHUNK
IB = 16
NC, NS = 2, 16
RPT = NPAD // NS
CPT0 = 128
CPT1 = 32
MAXPAIRS = max(CPT0, CPT1) // 2

_mesh = plsc.VectorSubcoreMesh(core_axis_name="c", subcore_axis_name="s")


@functools.partial(
    pl.kernel,
    out_type=jax.ShapeDtypeStruct((NC * NPAD,), jnp.float32),
    mesh=_mesh,
    scratch_types=[
        pltpu.VMEM((CPT, CHUNK), jnp.int32),
        pltpu.VMEM((RPT,), jnp.float32),
        pltpu.VMEM((CHUNK,), jnp.float32),
        pltpu.VMEM_SHARED((NPAD,), jnp.float32),
    ],
)
def _sc_degree(dst_hbm, out_hbm, idx_v, zeros_v, ones_v, acc):
    c = lax.axis_index("c")
    s = lax.axis_index("s")
    tile = c * NS + s
    pltpu.sync_copy(dst_hbm.at[pl.ds(pl.multiple_of(tile * CPT, 8), CPT)],
                    idx_v)

    def _fill(i, _):
        zeros_v[pl.ds(i * 16, 16)] = jnp.zeros((16,), jnp.float32)
        return 0

    lax.fori_loop(0, RPT // 16, _fill, 0)
    for j in range(CHUNK // 16):
        ones_v[pl.ds(j * 16, 16)] = jnp.ones((16,), jnp.float32)
    pltpu.sync_copy(zeros_v, acc.at[pl.ds(s * RPT, RPT)])
    plsc.subcore_barrier()

    def _body(j, _):
        pltpu.sync_copy(ones_v, acc.at[idx_v.at[j]], add=True)
        return 0

    lax.fori_loop(0, CPT, _body, 0)
    plsc.subcore_barrier()
    pltpu.sync_copy(acc.at[pl.ds(s * RPT, RPT)],
                    out_hbm.at[pl.ds(c * NPAD + s * RPT, RPT)])


@functools.partial(
    pl.kernel,
    out_type=jax.ShapeDtypeStruct((NC, NPAD, D), jnp.float32),
    mesh=_mesh,
    scratch_types=[
        pltpu.VMEM((2, IB, CHUNK), jnp.int32),
        pltpu.VMEM((2, IB, CHUNK), jnp.int32),
        pltpu.VMEM((2, CHUNK, D), jnp.float32),
        pltpu.VMEM_SHARED((NPAD, D), jnp.float32),
        pltpu.SemaphoreType.DMA,
        pltpu.SemaphoreType.DMA,
        pltpu.SemaphoreType.DMA,
        pltpu.SemaphoreType.DMA,
    ],
)
def _sc_adj_apply(y_hbm, src_hbm, dst_hbm, out_hbm,
                  srcb, dstb, rows_v, acc, g0, g1, ssem, isem):
    c = lax.axis_index("c")
    s = lax.axis_index("s")
    base = jnp.where(c == 0, s * CPT0, NS * CPT0 + s * CPT1)
    my_pairs = jnp.where(c == 0, CPT0 // 2, CPT1 // 2)

    def _iblk(nb):
        return pl.ds(pl.multiple_of(base + nb * IB, 8), IB)

    pltpu.sync_copy(src_hbm.at[_iblk(0)], srcb.at[0])
    pltpu.sync_copy(dst_hbm.at[_iblk(0)], dstb.at[0])

    def _fill(i, _):
        for j in range(D // 16):
            rows_v[0, i, pl.ds(j * 16, 16)] = jnp.zeros((16,), jnp.float32)
        return 0

    lax.fori_loop(0, CHUNK, _fill, 0)
    for k in range(RPT // CHUNK):
        pltpu.sync_copy(rows_v.at[0],
                        acc.at[pl.ds(s * RPT + k * CHUNK, CHUNK)])

    def _gather_start(blk, row, buf, sem):
        pltpu.async_copy(y_hbm.at[srcb.at[blk, row]], rows_v.at[buf], sem)

    def _gather_wait(blk, row, buf, sem):
        pltpu.make_async_copy(y_hbm.at[srcb.at[blk, row]], rows_v.at[buf],
                              sem).wait()

    _gather_start(0, 0, 0, g0)
    plsc.subcore_barrier()

    def _body(t, _):
        a = 2 * t
        blk = (a // IB) % 2
        ra = a % IB

        @pl.when(jnp.logical_and(t % (IB // 2) == 0, t < my_pairs - IB // 2))
        def _():
            nxt = t // (IB // 2) + 1
            pltpu.async_copy(src_hbm.at[_iblk(nxt)], srcb.at[nxt % 2], isem)
            pltpu.async_copy(dst_hbm.at[_iblk(nxt)], dstb.at[nxt % 2], isem)

        _gather_wait(blk, ra, 0, g0)
        _gather_start(blk, ra + 1, 1, g1)
        sd_a = pltpu.async_copy(rows_v.at[0], acc.at[dstb.at[blk, ra]],
                                ssem, add=True)
        _gather_wait(blk, ra + 1, 1, g1)
        sd_a.wait()

        @pl.when(t < my_pairs - 1)
        def _():
            n = a + 2
            nblk = (n // IB) % 2
            nr = n % IB

            @pl.when(nr == 0)
            def _():
                pltpu.make_async_copy(src_hbm.at[_iblk(0)],
                                      srcb.at[nblk], isem).wait()
                pltpu.make_async_copy(dst_hbm.at[_iblk(0)],
                                      dstb.at[nblk], isem).wait()

            _gather_start(nblk, nr, 0, g0)

        pltpu.sync_copy(rows_v.at[1], acc.at[dstb.at[blk, ra + 1]],
                        add=True)
        return 0

    lax.fori_loop(0, my_pairs, _body, 0)
    plsc.subcore_barrier()
    pltpu.sync_copy(acc.at[pl.ds(s * RPT, RPT)],
                    out_hbm.at[c, pl.ds(s * RPT, RPT)])


BM = 1024
GRID = NPAD // BM


def _dinv(degp_ref):
    return lax.rsqrt(jnp.maximum(degp_ref[0] + degp_ref[1], 1.0))


def _prep_body(x_ref, w_ref, degp_ref, o_ref):
    o_ref[...] = jnp.dot(x_ref[...], w_ref[...],
                         preferred_element_type=jnp.float32) * _dinv(degp_ref)


def _mid_body(p_ref, degp_ref, b_ref, w_ref, o_ref):
    dinv = _dinv(degp_ref)
    h = jnp.maximum((p_ref[0] + p_ref[1]) * dinv + b_ref[...], 0.0)
    o_ref[...] = jnp.dot(h, w_ref[...],
                         preferred_element_type=jnp.float32) * dinv


def _head_body(p_ref, degp_ref, b2_ref, wf1_ref, bf1_ref, wf2_ref, bf2_ref,
               o_ref):
    dinv = _dinv(degp_ref)
    h = jnp.maximum((p_ref[0] + p_ref[1]) * dinv + b2_ref[...], 0.0)
    f = jnp.maximum(jnp.dot(h, wf1_ref[...],
                            preferred_element_type=jnp.float32) + bf1_ref[...],
                    0.0)
    o_ref[...] = jnp.dot(f, wf2_ref[...],
                         preferred_element_type=jnp.float32) + bf2_ref[...]


_row_spec = pl.BlockSpec((BM, D), lambda i: (i, 0))
_degp_spec = pl.BlockSpec((NC, BM, 1), lambda i: (0, i, 0))
_p_spec = pl.BlockSpec((NC, BM, D), lambda i: (0, i, 0))
_w_spec = pl.BlockSpec((D, D), lambda i: (0, 0))
_b_spec = pl.BlockSpec((1, D), lambda i: (0, 0))

_tc_prep = pl.pallas_call(
    _prep_body,
    grid=(GRID,),
    in_specs=[_row_spec, _w_spec, _degp_spec],
    out_specs=_row_spec,
    out_shape=jax.ShapeDtypeStruct((NPAD, D), jnp.float32),
)

_tc_mid = pl.pallas_call(
    _mid_body,
    grid=(GRID,),
    in_specs=[_p_spec, _degp_spec, _b_spec, _w_spec],
    out_specs=_row_spec,
    out_shape=jax.ShapeDtypeStruct((NPAD, D), jnp.float32),
)

_tc_head = pl.pallas_call(
    _head_body,
    grid=(GRID,),
    in_specs=[_p_spec, _degp_spec, _b_spec, _w_spec, _b_spec,
              pl.BlockSpec((D, DOUT), lambda i: (0, 0)),
              pl.BlockSpec((1, DOUT), lambda i: (0, 0))],
    out_specs=pl.BlockSpec((BM, DOUT), lambda i: (i, 0)),
    out_shape=jax.ShapeDtypeStruct((NPAD, DOUT), jnp.float32),
)


def kernel(x, edge_index, W1, b1, W2, b2, Wf1, bf1, Wf2, bf2):
    src = edge_index[0].astype(jnp.int32)
    dst = edge_index[1].astype(jnp.int32)
    pad = jnp.full((EPAD - E,), N, jnp.int32)
    src2d = jnp.concatenate([src, pad]).reshape(NCHUNKS, CHUNK)
    dst2d = jnp.concatenate([dst, pad]).reshape(NCHUNKS, CHUNK)
    x_pad = jnp.zeros((NPAD, D), jnp.float32).at[:N].set(x)

    degp = _sc_degree(dst2d).reshape(NC, NPAD)[:, :, None]
    y1 = _tc_prep(x_pad, W1, degp)
    p1 = _sc_adj_apply(y1, src2d, dst2d)
    y2 = _tc_mid(p1, degp, b1.reshape(1, D), W2)
    p2 = _sc_adj_apply(y2, src2d, dst2d)
    out = _tc_head(p2, degp, b2.reshape(1, D), Wf1, bf1.reshape(1, D),
                   Wf2, bf2.reshape(1, DOUT))
    return out[:N]
